# baseline (speedup 1.0000x reference)
import numpy as np
import jax
import jax.numpy as jnp
from jax.experimental import pallas as pl
from jax.experimental.pallas import tpu as pltpu

BSZ = 128


def _make_body(b, h, t, dh):
    nb = t // BSZ
    scale = np.float32(dh ** -0.5)
    qscale = np.float32(dh ** -0.5 * np.log2(np.e))
    r = BSZ - 1
    gsz = min(8, nb)

    def body(w_ref, a_ref, q_ref, k_ref, v_ref, nk_ref, nv_ref,
             out_ref, idx_s, qst, kst, vst):
        stage = (qst, kst, vst)
        for sub in range(1):
            for src, dst in zip((q_ref, k_ref, v_ref), stage):
                dst[sub * t:sub * t + t - r, :] = src[0, 1, sub, r:t, :]
                dst[sub * t + t - r:(sub + 1) * t, :] = src[0, 1, sub, 0:r, :]

        def bucket(which, half, sub, u):
            if half:
                ref = stage[which]
                return ref[sub * t + u * BSZ:sub * t + (u + 1) * BSZ, :]
            ref = (q_ref, k_ref, v_ref)[which]
            return ref[0, 0, sub, u * BSZ:(u + 1) * BSZ, :]

        def gather(which, half, sub, offg):
            if half:
                return stage[which][pl.ds(sub * t + offg, BSZ), :]
            ref = (q_ref, k_ref, v_ref)[which]
            return ref[0, 0, sub, pl.ds(offg, BSZ), :]

        def prefix_excl(x):
            p = jnp.concatenate([jnp.zeros((1, dh), jnp.float32), x[:-1]],
                                axis=0)
            s = 1
            while s < nb:
                p = p + jnp.concatenate(
                    [jnp.zeros((s, dh), jnp.float32), p[:-s]], axis=0)
                s *= 2
            return p

        posn = (jax.lax.broadcasted_iota(jnp.int32, (nb, dh), 0) * BSZ
                + 1).astype(jnp.float32)
        ir = jax.lax.broadcasted_iota(jnp.int32, (nb, nb + 1), 0)
        jc = jax.lax.broadcasted_iota(jnp.int32, (nb, nb + 1), 1)

        mx_all, idx_rows = {}, []
        for half in range(2):
            for sub in range(1):
                qsums, ksums, kwsums, qfirsts = [], [], [], []
                for u in range(nb):
                    qb = bucket(0, half, sub, u)
                    kb = bucket(1, half, sub, u)
                    qsums.append(qb.sum(axis=0, keepdims=True))
                    ksums.append(kb.sum(axis=0, keepdims=True))
                    kwsums.append(
                        (kb * w_ref[u * BSZ:(u + 1) * BSZ, :]).sum(
                            axis=0, keepdims=True))
                    qfirsts.append(qb[0:1])

                qsum = jnp.concatenate(qsums, axis=0)
                ksum = jnp.concatenate(ksums, axis=0)
                kwsum = jnp.concatenate(kwsums, axis=0)
                qfirst = jnp.concatenate(qfirsts, axis=0)

                SQ = (prefix_excl(qsum) + qfirst) / posn
                sk = prefix_excl(ksum) * a_ref[:, :] + kwsum
                SK = jnp.concatenate(
                    [jnp.zeros((1, dh), jnp.float32), sk], axis=0)
                R = jax.lax.dot_general(SQ, SK, (((1,), (1,)), ((), ())),
                                        preferred_element_type=jnp.float32)
                Rm = jnp.where(jc <= ir, R * scale,
                               -jnp.finfo(jnp.float32).max)
                m = jnp.max(Rm, axis=1, keepdims=True)
                e = jnp.exp(Rm - m)
                p = e / jnp.sum(e, axis=1, keepdims=True)
                Rz = jnp.where(jc < ir, p, 0.0)
                mx = jnp.max(Rz, axis=1, keepdims=True)
                cand = jnp.where(Rz >= mx, jc.astype(jnp.float32),
                                 np.float32(1e9))
                idxf = jnp.min(cand, axis=1, keepdims=True)
                mx_all[(half, sub)] = mx
                idx_rows.append(jnp.broadcast_to(idxf.astype(jnp.int32),
                                                 (nb, BSZ)))
        idx_s[...] = jnp.concatenate(idx_rows, axis=0)

        nulls = {
            (half, sub): (
                jnp.broadcast_to(nk_ref[half, sub], (BSZ, dh)),
                jnp.broadcast_to(nv_ref[half, sub], (BSZ, dh)))
            for half in range(2) for sub in range(1)
        }

        ii = jax.lax.broadcasted_iota(jnp.int32, (BSZ, BSZ), 0)
        jj = jax.lax.broadcasted_iota(jnp.int32, (BSZ, BSZ), 1)
        own_causal = jj <= ii

        def flush(groups, u):
            base = (u - gsz + 1) * BSZ
            ow0 = jnp.stack([jnp.concatenate(groups[(0, s)], axis=0)
                             for s in range(1)], axis=0)
            ow1 = jnp.stack([jnp.concatenate(groups[(1, s)], axis=0)
                             for s in range(1)], axis=0)
            out_ref[0, 0, :, base:base + gsz * BSZ, :] = ow0
            if base + gsz * BSZ == t:
                out_ref[0, 1, :, base + r:t, :] = ow1[:, 0:gsz * BSZ - r, :]
                out_ref[0, 1, :, 0:r, :] = ow1[:, gsz * BSZ - r:, :]
            else:
                out_ref[0, 1, :, base + r:base + gsz * BSZ + r, :] = ow1

        groups = {(hf, s): [] for hf in range(2) for s in range(1)}
        for u in range(nb):
            for half in range(2):
                for sub in range(1):
                    qu = bucket(0, half, sub, u)
                    ku = bucket(1, half, sub, u)
                    vu = bucket(2, half, sub, u)
                    qu16 = (qu * qscale).astype(jnp.bfloat16)
                    vu16 = vu.astype(jnp.bfloat16)

                    dots_own = jax.lax.dot_general(
                        qu16, ku.astype(jnp.bfloat16),
                        (((1,), (1,)), ((), ())),
                        preferred_element_type=jnp.float32)
                    e_own = jnp.exp2(dots_own)
                    if u == nb - 1 and half:
                        own_mask = own_causal & ~((jj == 0) & (ii >= 1))
                    else:
                        own_mask = own_causal
                    e_own = jnp.where(own_mask, e_own, 0.0)

                    if u == 0:
                        acc = jax.lax.dot_general(
                            e_own.astype(jnp.bfloat16), vu16,
                            (((1,), (0,)), ((), ())),
                            preferred_element_type=jnp.float32)
                        s = jnp.sum(e_own, axis=1, keepdims=True) \
                            + np.float32(BSZ)
                        groups[(half, sub)].append(acc / s)
                        continue

                    iu = idx_s[(half + sub) * nb + u, 0]
                    offg = pl.multiple_of(
                        jnp.maximum(iu - 1, 0) * BSZ, BSZ)
                    kgl = gather(1, half, sub, offg)
                    vgl = gather(2, half, sub, offg)
                    isnull = iu == 0
                    kg = jnp.where(isnull, nulls[(half, sub)][0], kgl)
                    vg = jnp.where(isnull, nulls[(half, sub)][1], vgl)
                    val = mx_all[(half, sub)][u:u + 1, 0:1]

                    dots_g = jax.lax.dot_general(
                        qu16, (kg * val).astype(jnp.bfloat16),
                        (((1,), (1,)), ((), ())),
                        preferred_element_type=jnp.float32)
                    e_g = jnp.exp2(dots_g)
                    if u == nb - 1 and half:
                        e_g = jnp.where(ii == 0, e_g, 0.0)

                    acc = jax.lax.dot_general(
                        e_g.astype(jnp.bfloat16),
                        (vg * val).astype(jnp.bfloat16),
                        (((1,), (0,)), ((), ())),
                        preferred_element_type=jnp.float32) \
                        + jax.lax.dot_general(
                            e_own.astype(jnp.bfloat16), vu16,
                            (((1,), (0,)), ((), ())),
                            preferred_element_type=jnp.float32)
                    s = (jnp.sum(e_g, axis=1, keepdims=True)
                         + jnp.sum(e_own, axis=1, keepdims=True))
                    groups[(half, sub)].append(acc / s)
            if len(groups[(0, 0)]) == gsz:
                flush(groups, u)
                groups = {(hf, s): [] for hf in range(2) for s in range(1)}

    return body


def kernel(q, k, v, null_keys, null_values):
    b, h, t, dh = q.shape
    nb = t // BSZ
    hh = h // 2
    nhalf = hh

    pos = np.arange(t, dtype=np.float64).reshape(nb, BSZ)
    wmat = 1.0 / (pos + 1.0)
    wsuf = np.cumsum(wmat[:, ::-1], axis=1)[:, ::-1]
    wfull = jnp.asarray(
        np.broadcast_to(wsuf.reshape(t, 1), (t, dh)), dtype=jnp.float32)
    afull = jnp.asarray(
        np.broadcast_to(wsuf[:, 0:1], (nb, dh)), dtype=jnp.float32)

    q5 = q.reshape(b, 2, hh, t, dh)
    k5 = k.reshape(b, 2, hh, t, dh)
    v5 = v.reshape(b, 2, hh, t, dh)
    nk4 = null_keys.reshape(2, hh, 1, dh)
    nv4 = null_values.reshape(2, hh, 1, dh)

    def rowmap(i):
        return (i // nhalf, 0, i % nhalf, 0, 0)

    fused = pl.pallas_call(
        _make_body(b, h, t, dh),
        grid=(b * nhalf,),
        in_specs=[
            pl.BlockSpec((t, dh), lambda i: (0, 0)),
            pl.BlockSpec((nb, dh), lambda i: (0, 0)),
            pl.BlockSpec((1, 2, 1, t, dh), rowmap),
            pl.BlockSpec((1, 2, 1, t, dh), rowmap),
            pl.BlockSpec((1, 2, 1, t, dh), rowmap),
            pl.BlockSpec((2, 1, 1, dh), lambda i: (0, i % nhalf, 0, 0)),
            pl.BlockSpec((2, 1, 1, dh), lambda i: (0, i % nhalf, 0, 0)),
        ],
        out_specs=pl.BlockSpec((1, 2, 1, t, dh), rowmap),
        out_shape=jax.ShapeDtypeStruct((b, 2, hh, t, dh), jnp.float32),
        scratch_shapes=[
            pltpu.VMEM((2 * nb, BSZ), jnp.int32),
            pltpu.VMEM((t, dh), jnp.float32),
            pltpu.VMEM((t, dh), jnp.float32),
            pltpu.VMEM((t, dh), jnp.float32),
        ],
        compiler_params=pltpu.CompilerParams(
            dimension_semantics=("arbitrary",)),
    )
    out5 = fused(wfull, afull, q5, k5, v5, nk4, nv4)
    return out5.reshape(b, h, t, dh)
